# speedup vs baseline: 4.0854x; 4.0854x over previous
import functools

import jax
import jax.numpy as jnp
from jax import lax
from jax.experimental import pallas as pl
from jax.experimental.pallas import tpu as pltpu
from jax.experimental.pallas import tpu_sc as plsc

VOCAB = 100000
EMB = 300
NUM_CLASSES = 128
BATCH = 4096
SEQ = 200

NC = 2
NS = 16
NW = NC * NS
B_PER_W = BATCH // NW
HALF = SEQ // 2
NVR = NUM_CLASSES // 16


def _proj_body(t_ref, w_ref, o_ref):
    o_ref[...] = lax.dot_general(
        t_ref[...], w_ref[...] * (1.0 / SEQ),
        (((1,), (0,)), ((), ())),
        preferred_element_type=jnp.float32,
    )


_ROWS_BLK = 2000


@jax.jit
def _project(table, W):
    return pl.pallas_call(
        _proj_body,
        grid=(VOCAB // _ROWS_BLK,),
        in_specs=[
            pl.BlockSpec((_ROWS_BLK, EMB), lambda i: (i, 0)),
            pl.BlockSpec((EMB, NUM_CLASSES), lambda i: (0, 0)),
        ],
        out_specs=pl.BlockSpec((_ROWS_BLK, NUM_CLASSES), lambda i: (i, 0)),
        out_shape=jax.ShapeDtypeStruct((VOCAB, NUM_CLASSES), jnp.float32),
    )(table, W)


def _pool_body(x_hbm, proj_hbm, bias_hbm, out_hbm,
               idx_v, rows_a, rows_b, bias_v, acc_v, sem_a, sem_b):
    cid = lax.axis_index("c")
    sid = lax.axis_index("s")
    wid = sid * NC + cid
    base2 = wid * (2 * B_PER_W)

    pltpu.sync_copy(x_hbm.at[pl.ds(base2, 2 * B_PER_W)], idx_v)
    pltpu.sync_copy(bias_hbm, bias_v)

    def elem(i, carry):
        cpa = pltpu.async_copy(proj_hbm.at[idx_v.at[2 * i]], rows_a, sem_a)
        cpb = pltpu.async_copy(proj_hbm.at[idx_v.at[2 * i + 1]], rows_b, sem_b)
        cpa.wait()
        cpb.wait()

        def red_a(s, accs):
            return tuple(accs[v] + rows_a[s, pl.ds(16 * v, 16)]
                         for v in range(NVR))

        def red_b(s, accs):
            return tuple(accs[v] + rows_b[s, pl.ds(16 * v, 16)]
                         for v in range(NVR))

        accs = tuple(jnp.zeros((16,), jnp.float32) for _ in range(NVR))
        accs = lax.fori_loop(0, HALF, red_a, accs)
        accs = lax.fori_loop(0, HALF, red_b, accs)
        for v in range(NVR):
            z = accs[v] + bias_v[pl.ds(16 * v, 16)]
            acc_v[i, pl.ds(16 * v, 16)] = 1.0 / (1.0 + jnp.exp(-z))
        return carry

    lax.fori_loop(0, B_PER_W, elem, 0)
    pltpu.sync_copy(acc_v, out_hbm.at[pl.ds(wid * B_PER_W, B_PER_W)])


@jax.jit
def _pool(x2, proj, b):
    mesh = plsc.VectorSubcoreMesh(
        core_axis_name="c", subcore_axis_name="s",
        num_cores=NC, num_subcores=NS,
    )
    f = pl.kernel(
        _pool_body,
        out_type=jax.ShapeDtypeStruct((BATCH, NUM_CLASSES), jnp.float32),
        mesh=mesh,
        scratch_types=[
            pltpu.VMEM((2 * B_PER_W, HALF), jnp.int32),
            pltpu.VMEM((HALF, NUM_CLASSES), jnp.float32),
            pltpu.VMEM((HALF, NUM_CLASSES), jnp.float32),
            pltpu.VMEM((NUM_CLASSES,), jnp.float32),
            pltpu.VMEM((B_PER_W, NUM_CLASSES), jnp.float32),
            pltpu.SemaphoreType.DMA,
            pltpu.SemaphoreType.DMA,
        ],
    )
    return f(x2, proj, b)


def kernel(x, table, W, b):
    proj = _project(table, W)
    x2 = jnp.reshape(x.astype(jnp.int32), (2 * BATCH, HALF))
    return _pool(x2, proj, b)


# speedup vs baseline: 5.6003x; 1.3708x over previous
import functools

import jax
import jax.numpy as jnp
from jax import lax
from jax.experimental import pallas as pl
from jax.experimental.pallas import tpu as pltpu
from jax.experimental.pallas import tpu_sc as plsc

VOCAB = 100000
EMB = 300
NUM_CLASSES = 128
BATCH = 4096
SEQ = 200

NC = 2
NS = 16
NW = NC * NS
B_PER_W = BATCH // NW
HALF = SEQ // 2
NVR = NUM_CLASSES // 16


def _proj_body(t_ref, w_ref, o_ref):
    o_ref[...] = lax.dot_general(
        t_ref[...], w_ref[...] * (1.0 / SEQ),
        (((1,), (0,)), ((), ())),
        preferred_element_type=jnp.float32,
    )


_ROWS_BLK = 2000


@jax.jit
def _project(table, W):
    return pl.pallas_call(
        _proj_body,
        grid=(VOCAB // _ROWS_BLK,),
        in_specs=[
            pl.BlockSpec((_ROWS_BLK, EMB), lambda i: (i, 0)),
            pl.BlockSpec((EMB, NUM_CLASSES), lambda i: (0, 0)),
        ],
        out_specs=pl.BlockSpec((_ROWS_BLK, NUM_CLASSES), lambda i: (i, 0)),
        out_shape=jax.ShapeDtypeStruct((VOCAB, NUM_CLASSES), jnp.float32),
    )(table, W)


def _pool_body(x_hbm, proj_hbm, bias_hbm, out_hbm,
               idx_v, r0a, r0b, r1a, r1b, bias_v, acc_v,
               s0a, s0b, s1a, s1b):
    cid = lax.axis_index("c")
    sid = lax.axis_index("s")
    wid = sid * NC + cid
    base2 = wid * (2 * B_PER_W)

    pltpu.sync_copy(x_hbm.at[pl.ds(base2, 2 * B_PER_W)], idx_v)
    pltpu.sync_copy(bias_hbm, bias_v)

    def fire(e, ra, rb, sa, sb):
        pltpu.async_copy(proj_hbm.at[idx_v.at[2 * e]], ra, sa)
        pltpu.async_copy(proj_hbm.at[idx_v.at[2 * e + 1]], rb, sb)

    def drain(ra, rb, sa, sb):
        pltpu.make_async_copy(proj_hbm.at[idx_v.at[0]], ra, sa).wait()
        pltpu.make_async_copy(proj_hbm.at[idx_v.at[0]], rb, sb).wait()

    def reduce_and_store(e, ra, rb):
        def red(rref):
            def body(s, accs):
                return tuple(a + rref[s, pl.ds(16 * v, 16)]
                             for v, a in enumerate(accs))
            return body

        accs = tuple(jnp.zeros((16,), jnp.float32) for _ in range(NVR))
        accs = lax.fori_loop(0, HALF, red(ra), accs)
        accs = lax.fori_loop(0, HALF, red(rb), accs)
        for v in range(NVR):
            z = accs[v] + bias_v[pl.ds(16 * v, 16)]
            acc_v[e, pl.ds(16 * v, 16)] = 1.0 / (1.0 + jnp.exp(-z))

    fire(0, r0a, r0b, s0a, s0b)

    def pair(i, carry):
        e0 = 2 * i
        fire(e0 + 1, r1a, r1b, s1a, s1b)
        drain(r0a, r0b, s0a, s0b)
        reduce_and_store(e0, r0a, r0b)

        @pl.when(i < B_PER_W // 2 - 1)
        def _():
            fire(e0 + 2, r0a, r0b, s0a, s0b)

        drain(r1a, r1b, s1a, s1b)
        reduce_and_store(e0 + 1, r1a, r1b)
        return carry

    lax.fori_loop(0, B_PER_W // 2, pair, 0)
    pltpu.sync_copy(acc_v, out_hbm.at[pl.ds(wid * B_PER_W, B_PER_W)])


@jax.jit
def _pool(x2, proj, b):
    mesh = plsc.VectorSubcoreMesh(
        core_axis_name="c", subcore_axis_name="s",
        num_cores=NC, num_subcores=NS,
    )
    f = pl.kernel(
        _pool_body,
        out_type=jax.ShapeDtypeStruct((BATCH, NUM_CLASSES), jnp.float32),
        mesh=mesh,
        scratch_types=[
            pltpu.VMEM((2 * B_PER_W, HALF), jnp.int32),
            pltpu.VMEM((HALF, NUM_CLASSES), jnp.float32),
            pltpu.VMEM((HALF, NUM_CLASSES), jnp.float32),
            pltpu.VMEM((HALF, NUM_CLASSES), jnp.float32),
            pltpu.VMEM((HALF, NUM_CLASSES), jnp.float32),
            pltpu.VMEM((NUM_CLASSES,), jnp.float32),
            pltpu.VMEM((B_PER_W, NUM_CLASSES), jnp.float32),
            pltpu.SemaphoreType.DMA,
            pltpu.SemaphoreType.DMA,
            pltpu.SemaphoreType.DMA,
            pltpu.SemaphoreType.DMA,
        ],
    )
    return f(x2, proj, b)


def kernel(x, table, W, b):
    proj = _project(table, W)
    x2 = jnp.reshape(x.astype(jnp.int32), (2 * BATCH, HALF))
    return _pool(x2, proj, b)


# speedup vs baseline: 5.7806x; 1.0322x over previous
import functools

import jax
import jax.numpy as jnp
from jax import lax
from jax.experimental import pallas as pl
from jax.experimental.pallas import tpu as pltpu
from jax.experimental.pallas import tpu_sc as plsc

VOCAB = 100000
EMB = 300
NUM_CLASSES = 128
BATCH = 4096
SEQ = 200

NC = 2
NS = 16
NW = NC * NS
B_PER_W = BATCH // NW
HALF = SEQ // 2
NVR = NUM_CLASSES // 16


def _proj_body(t_ref, w_ref, o_ref):
    o_ref[...] = lax.dot_general(
        t_ref[...], w_ref[...] * (1.0 / SEQ),
        (((1,), (0,)), ((), ())),
        preferred_element_type=jnp.float32,
    )


_ROWS_BLK = 5000


@jax.jit
def _project(table, W):
    return pl.pallas_call(
        _proj_body,
        grid=(VOCAB // _ROWS_BLK,),
        in_specs=[
            pl.BlockSpec((_ROWS_BLK, EMB), lambda i: (i, 0)),
            pl.BlockSpec((EMB, NUM_CLASSES), lambda i: (0, 0)),
        ],
        out_specs=pl.BlockSpec((_ROWS_BLK, NUM_CLASSES), lambda i: (i, 0)),
        out_shape=jax.ShapeDtypeStruct((VOCAB, NUM_CLASSES), jnp.float32),
    )(table, W)


def _pool_body(x_hbm, proj_hbm, bias_hbm, out_hbm,
               idx_v, r0a, r0b, r1a, r1b, bias_v, acc_v,
               s0a, s0b, s1a, s1b):
    cid = lax.axis_index("c")
    sid = lax.axis_index("s")
    wid = sid * NC + cid
    base2 = wid * (2 * B_PER_W)

    pltpu.sync_copy(x_hbm.at[pl.ds(base2, 2 * B_PER_W)], idx_v)
    pltpu.sync_copy(bias_hbm, bias_v)

    def fire(e, ra, rb, sa, sb):
        pltpu.async_copy(proj_hbm.at[idx_v.at[2 * e]], ra, sa)
        pltpu.async_copy(proj_hbm.at[idx_v.at[2 * e + 1]], rb, sb)

    def drain(ra, rb, sa, sb):
        pltpu.make_async_copy(proj_hbm.at[idx_v.at[0]], ra, sa).wait()
        pltpu.make_async_copy(proj_hbm.at[idx_v.at[0]], rb, sb).wait()

    def reduce_and_store(e, ra, rb):
        def red(rref):
            def body(s, accs):
                return tuple(a + rref[s, pl.ds(16 * v, 16)]
                             for v, a in enumerate(accs))
            return body

        accs = tuple(jnp.zeros((16,), jnp.float32) for _ in range(NVR))
        accs = lax.fori_loop(0, HALF, red(ra), accs)
        accs = lax.fori_loop(0, HALF, red(rb), accs)
        for v in range(NVR):
            z = accs[v] + bias_v[pl.ds(16 * v, 16)]
            acc_v[e, pl.ds(16 * v, 16)] = 1.0 / (1.0 + jnp.exp(-z))

    fire(0, r0a, r0b, s0a, s0b)

    def pair(i, carry):
        e0 = 2 * i
        fire(e0 + 1, r1a, r1b, s1a, s1b)
        drain(r0a, r0b, s0a, s0b)
        reduce_and_store(e0, r0a, r0b)

        @pl.when(i < B_PER_W // 2 - 1)
        def _():
            fire(e0 + 2, r0a, r0b, s0a, s0b)

        drain(r1a, r1b, s1a, s1b)
        reduce_and_store(e0 + 1, r1a, r1b)
        return carry

    lax.fori_loop(0, B_PER_W // 2, pair, 0)
    pltpu.sync_copy(acc_v, out_hbm.at[pl.ds(wid * B_PER_W, B_PER_W)])


@jax.jit
def _pool(x2, proj, b):
    mesh = plsc.VectorSubcoreMesh(
        core_axis_name="c", subcore_axis_name="s",
        num_cores=NC, num_subcores=NS,
    )
    f = pl.kernel(
        _pool_body,
        out_type=jax.ShapeDtypeStruct((BATCH, NUM_CLASSES), jnp.float32),
        mesh=mesh,
        scratch_types=[
            pltpu.VMEM((2 * B_PER_W, HALF), jnp.int32),
            pltpu.VMEM((HALF, NUM_CLASSES), jnp.float32),
            pltpu.VMEM((HALF, NUM_CLASSES), jnp.float32),
            pltpu.VMEM((HALF, NUM_CLASSES), jnp.float32),
            pltpu.VMEM((HALF, NUM_CLASSES), jnp.float32),
            pltpu.VMEM((NUM_CLASSES,), jnp.float32),
            pltpu.VMEM((B_PER_W, NUM_CLASSES), jnp.float32),
            pltpu.SemaphoreType.DMA,
            pltpu.SemaphoreType.DMA,
            pltpu.SemaphoreType.DMA,
            pltpu.SemaphoreType.DMA,
        ],
    )
    return f(x2, proj, b)


def kernel(x, table, W, b):
    proj = _project(table, W)
    x2 = jnp.reshape(x.astype(jnp.int32), (2 * BATCH, HALF))
    return _pool(x2, proj, b)
